# baseline (speedup 1.0000x reference)
import functools

import jax
import jax.numpy as jnp
from jax import lax
from jax.experimental import pallas as pl
from jax.experimental.pallas import tpu as pltpu
from jax.experimental.pallas import tpu_sc as plsc

NUM_I = 4096
NUM_J = 200
NUM_INDICES = NUM_I * NUM_J
DIM = 64
PAD_DIM = 128
LANES = 16
NUM_CORES = 2
NUM_SUBCORES = 16
NUM_WORKERS = NUM_CORES * NUM_SUBCORES
PER_WORKER = NUM_INDICES // NUM_WORKERS
CHUNK = 128
IBLKS = NUM_I // CHUNK
NUM_CHUNKS = PER_WORKER // CHUNK
NGBUF = 4
NTBUF = 2

_mesh = plsc.VectorSubcoreMesh(core_axis_name="c", subcore_axis_name="s")


@functools.partial(
    pl.kernel,
    mesh=_mesh,
    compiler_params=pltpu.CompilerParams(needs_layout_passes=False),
    out_type=jax.ShapeDtypeStruct((NUM_J, DIM, NUM_I), jnp.float32),
    scratch_types=[
        pltpu.VMEM((PER_WORKER,), jnp.int32),
        [pltpu.VMEM((CHUNK, PAD_DIM), jnp.float32) for _ in range(NGBUF)],
        [pltpu.VMEM((DIM, CHUNK), jnp.float32) for _ in range(NTBUF)],
        [pltpu.SemaphoreType.DMA for _ in range(NGBUF)],
        [pltpu.SemaphoreType.DMA for _ in range(NTBUF)],
    ],
)
def _gather_kernel(idx_hbm, table_hbm, out_hbm, idx_v, rows, trans, gsems, ssems):
    wid = lax.axis_index("s") * NUM_CORES + lax.axis_index("c")
    base = wid * NUM_CHUNKS

    def start_gather(g, gb):
        pltpu.async_copy(
            table_hbm.at[idx_v.at[pl.ds(g * CHUNK, CHUNK)]], rows[gb], gsems[gb]
        )

    def wait_gather(gb):
        pltpu.make_async_copy(
            table_hbm.at[idx_v.at[pl.ds(0, CHUNK)]], rows[gb], gsems[gb]
        ).wait()

    def transpose(gb, tb):
        lanes = lax.iota(jnp.int32, LANES)

        def col4(k4, carry):
            for u in range(4):
                k = k4 * 4 + u
                cols = jnp.broadcast_to(k, (LANES,))
                for mb in range(CHUNK // LANES):
                    vals = plsc.load_gather(rows[gb], [mb * LANES + lanes, cols])
                    trans[tb][k, pl.ds(mb * LANES, LANES)] = vals
            return carry

        lax.fori_loop(0, DIM // 4, col4, 0)

    def start_store(g, tb):
        gg = base + g
        j = gg // IBLKS
        iblk = gg - j * IBLKS
        pltpu.async_copy(
            trans[tb], out_hbm.at[j, :, pl.ds(iblk * CHUNK, CHUNK)], ssems[tb]
        )

    def wait_store(tb):
        pltpu.make_async_copy(
            trans[tb], out_hbm.at[0, :, pl.ds(0, CHUNK)], ssems[tb]
        ).wait()

    def slot(g, gb, tb, first, last):
        if not first:
            wait_store(tb)
        wait_gather(gb)
        transpose(gb, tb)
        start_store(g, tb)
        if not last:
            start_gather(g + NGBUF, gb)

    pltpu.sync_copy(idx_hbm.at[pl.ds(wid * PER_WORKER, PER_WORKER)], idx_v)

    for gb in range(NGBUF):
        start_gather(gb, gb)

    for g in (0, 1):
        slot(g, g % NGBUF, g % NTBUF, first=True, last=False)

    def body(t, carry):
        for p in range(4):
            g = 2 + t * 4 + p
            slot(g, (2 + p) % NGBUF, p % NTBUF, first=False, last=False)
        return carry

    lax.fori_loop(0, (NUM_CHUNKS - NGBUF - 4) // 4, body, 0)

    for g in range(NUM_CHUNKS - NGBUF - 2, NUM_CHUNKS - NGBUF):
        slot(g, g % NGBUF, g % NTBUF, first=False, last=False)

    for g in range(NUM_CHUNKS - NGBUF, NUM_CHUNKS):
        slot(g, g % NGBUF, g % NTBUF, first=False, last=True)

    for tb in range(NTBUF):
        wait_store(tb)


def kernel(timesteps, table):
    idx = jnp.swapaxes(timesteps, 0, 1).reshape(-1).astype(jnp.int32)
    table_p = jnp.pad(table, ((0, 0), (0, PAD_DIM - DIM)))
    out_p = _gather_kernel(idx, table_p)
    return jnp.transpose(out_p, (2, 0, 1))


# speedup vs baseline: 2.7435x; 2.7435x over previous
import functools

import jax
import jax.numpy as jnp
from jax import lax
from jax.experimental import pallas as pl
from jax.experimental.pallas import tpu as pltpu
from jax.experimental.pallas import tpu_sc as plsc

NUM_I = 4096
NUM_J = 200
NUM_INDICES = NUM_I * NUM_J
DIM = 64
PAD_DIM = 128
LANES = 16
NUM_CORES = 2
NUM_SUBCORES = 16
NUM_WORKERS = NUM_CORES * NUM_SUBCORES
PER_WORKER = NUM_INDICES // NUM_WORKERS
CHUNK = 128
IBLKS = NUM_I // CHUNK
NUM_CHUNKS = PER_WORKER // CHUNK
NGBUF = 4
NTBUF = 2

_mesh = plsc.VectorSubcoreMesh(core_axis_name="c", subcore_axis_name="s")


@functools.partial(
    pl.kernel,
    mesh=_mesh,
    compiler_params=pltpu.CompilerParams(needs_layout_passes=False),
    out_type=jax.ShapeDtypeStruct((NUM_J, DIM, NUM_I), jnp.float32),
    scratch_types=[
        pltpu.VMEM((PER_WORKER,), jnp.int32),
        [pltpu.VMEM((CHUNK, PAD_DIM), jnp.float32) for _ in range(NGBUF)],
        [pltpu.VMEM((DIM, CHUNK), jnp.float32) for _ in range(NTBUF)],
        [pltpu.SemaphoreType.DMA for _ in range(NGBUF)],
        [pltpu.SemaphoreType.DMA for _ in range(NTBUF)],
    ],
)
def _gather_kernel(idx_hbm, table_hbm, out_hbm, idx_v, rows, trans, gsems, ssems):
    wid = lax.axis_index("s") * NUM_CORES + lax.axis_index("c")
    base = wid * NUM_CHUNKS

    def start_gather(g, gb):
        pltpu.async_copy(
            table_hbm.at[idx_v.at[pl.ds(g * CHUNK, CHUNK)]], rows[gb], gsems[gb]
        )

    def wait_gather(gb):
        pltpu.make_async_copy(
            table_hbm.at[idx_v.at[pl.ds(0, CHUNK)]], rows[gb], gsems[gb]
        ).wait()

    def transpose(gb, tb):
        lanes = lax.iota(jnp.int32, LANES)

        def diag(d, carry):
            for mb in range(CHUNK // LANES):
                rowv = mb * LANES + ((lanes + d) & (LANES - 1))
                for kb in range(DIM // LANES):
                    colv = kb * LANES + lanes
                    vals = plsc.load_gather(rows[gb], [rowv, colv])
                    plsc.store_scatter(trans[tb], [colv, rowv], vals)
            return carry

        lax.fori_loop(0, LANES, diag, 0)

    def start_store(g, tb):
        gg = base + g
        j = gg // IBLKS
        iblk = gg - j * IBLKS
        pltpu.async_copy(
            trans[tb], out_hbm.at[j, :, pl.ds(iblk * CHUNK, CHUNK)], ssems[tb]
        )

    def wait_store(tb):
        pltpu.make_async_copy(
            trans[tb], out_hbm.at[0, :, pl.ds(0, CHUNK)], ssems[tb]
        ).wait()

    def slot(g, gb, tb, first, last):
        if not first:
            wait_store(tb)
        wait_gather(gb)
        transpose(gb, tb)
        start_store(g, tb)
        if not last:
            start_gather(g + NGBUF, gb)

    pltpu.sync_copy(idx_hbm.at[pl.ds(wid * PER_WORKER, PER_WORKER)], idx_v)

    for gb in range(NGBUF):
        start_gather(gb, gb)

    for g in (0, 1):
        slot(g, g % NGBUF, g % NTBUF, first=True, last=False)

    def body(t, carry):
        for p in range(4):
            g = 2 + t * 4 + p
            slot(g, (2 + p) % NGBUF, p % NTBUF, first=False, last=False)
        return carry

    lax.fori_loop(0, (NUM_CHUNKS - NGBUF - 4) // 4, body, 0)

    for g in range(NUM_CHUNKS - NGBUF - 2, NUM_CHUNKS - NGBUF):
        slot(g, g % NGBUF, g % NTBUF, first=False, last=False)

    for g in range(NUM_CHUNKS - NGBUF, NUM_CHUNKS):
        slot(g, g % NGBUF, g % NTBUF, first=False, last=True)

    for tb in range(NTBUF):
        wait_store(tb)


def kernel(timesteps, table):
    idx = jnp.swapaxes(timesteps, 0, 1).reshape(-1).astype(jnp.int32)
    table_p = jnp.pad(table, ((0, 0), (0, PAD_DIM - DIM)))
    out_p = _gather_kernel(idx, table_p)
    return jnp.transpose(out_p, (2, 0, 1))
